# baseline (speedup 1.0000x reference)
import functools

import jax
import jax.numpy as jnp
from jax import lax
from jax.experimental import pallas as pl
from jax.experimental.pallas import tpu as pltpu
from jax.experimental.pallas import tpu_sc as plsc

SEQ_LEN = 50
BATCH = 16384
D_MODEL = 64
DP = 2 * D_MODEL
VOCAB = 1000000
TOTAL = SEQ_LEN * BATCH

NUM_CORES = 2
NUM_SUBCORES = 16
NUM_WORKERS = NUM_CORES * NUM_SUBCORES

_mesh = plsc.VectorSubcoreMesh(core_axis_name="c", subcore_axis_name="s")

CH_T = 128
NFULL = VOCAB // CH_T
REM = VOCAB - NFULL * CH_T
K_T = NFULL // NUM_WORKERS + 1


def _transpose_block(src_v, dst_v):
    iota = lax.iota(jnp.int32, 16)
    for d0 in range(0, D_MODEL, 16):
        d_idx = iota + d0
        for v in range(CH_T):
            vec = plsc.load_gather(src_v, [d_idx, jnp.full((16,), v, jnp.int32)])
            dst_v[v, pl.ds(d0, 16)] = vec


@functools.partial(
    pl.kernel,
    mesh=_mesh,
    out_type=jax.ShapeDtypeStruct((VOCAB, DP), jnp.float32),
    scratch_types=[
        pltpu.VMEM((D_MODEL, CH_T), jnp.float32),
        pltpu.VMEM((CH_T, DP), jnp.float32),
    ],
    compiler_params=pltpu.CompilerParams(
        use_tc_tiling_on_sc=True, needs_layout_passes=False
    ),
)
def _table_transpose(tab_t_hbm, rem_hbm, out_hbm, buf_in, buf_out):
    wid = lax.axis_index("s") * NUM_CORES + lax.axis_index("c")

    def body(k, carry):
        c = wid + k * NUM_WORKERS

        @pl.when(c < NFULL)
        def _():
            v0 = c * CH_T
            pltpu.sync_copy(tab_t_hbm.at[:, pl.ds(v0, CH_T)], buf_in)
            _transpose_block(buf_in, buf_out)
            pltpu.sync_copy(buf_out, out_hbm.at[pl.ds(v0, CH_T)])

        @pl.when(c == NFULL)
        def _():
            pltpu.sync_copy(rem_hbm, buf_out.at[pl.ds(0, REM)])
            pltpu.sync_copy(
                buf_out.at[pl.ds(0, REM)], out_hbm.at[pl.ds(NFULL * CH_T, REM)]
            )

        return carry

    lax.fori_loop(0, K_T, body, 0)


COLS = BATCH // NUM_WORKERS
CHUNK = 128
N_SUB = COLS // CHUNK


@functools.partial(
    pl.kernel,
    mesh=_mesh,
    out_type=jax.ShapeDtypeStruct((SEQ_LEN, BATCH, DP), jnp.float32),
    scratch_types=[
        pltpu.VMEM((COLS,), jnp.int32),
        pltpu.VMEM((COLS, DP), jnp.float32),
        pltpu.SemaphoreType.DMA,
    ],
    compiler_params=pltpu.CompilerParams(use_tc_tiling_on_sc=True),
)
def _embed_gather(idx_hbm, table_hbm, out_hbm, idx_v, rows_v, sem):
    wid = lax.axis_index("s") * NUM_CORES + lax.axis_index("c")
    col0 = wid * COLS

    def body(s, carry):
        pltpu.sync_copy(idx_hbm.at[pl.ds(s * BATCH + col0, COLS)], idx_v)
        copies = [
            pltpu.async_copy(
                table_hbm.at[idx_v.at[pl.ds(k * CHUNK, CHUNK)]],
                rows_v.at[pl.ds(k * CHUNK, CHUNK)],
                sem,
            )
            for k in range(N_SUB)
        ]
        for c in copies:
            c.wait()
        pltpu.sync_copy(rows_v, out_hbm.at[s, pl.ds(col0, COLS)])
        return carry

    lax.fori_loop(0, SEQ_LEN, body, 0)


def kernel(input, table):
    rem_p = jnp.pad(table[NFULL * CH_T :, :], ((0, 0), (0, DP - D_MODEL)))
    tbl_pad = _table_transpose(table.T, rem_p)
    out128 = _embed_gather(input.reshape(TOTAL), tbl_pad)
    return out128[:, :, :D_MODEL]


# speedup vs baseline: 1.5514x; 1.5514x over previous
import functools

import jax
import jax.numpy as jnp
from jax import lax
from jax.experimental import pallas as pl
from jax.experimental.pallas import tpu as pltpu
from jax.experimental.pallas import tpu_sc as plsc

SEQ_LEN = 50
BATCH = 16384
D_MODEL = 64
DP = 2 * D_MODEL
VOCAB = 1000000
TOTAL = SEQ_LEN * BATCH

NUM_CORES = 2
NUM_SUBCORES = 16
NUM_WORKERS = NUM_CORES * NUM_SUBCORES

_mesh = plsc.VectorSubcoreMesh(core_axis_name="c", subcore_axis_name="s")

CH_T = 128
NFULL = VOCAB // CH_T
REM = VOCAB - NFULL * CH_T
K_T = NFULL // NUM_WORKERS + 1


def _transpose_block(src_v, dst_v):
    iota = lax.iota(jnp.int32, 16)
    io2 = iota >> 1
    iop64 = (iota & 1) * D_MODEL
    row_g = [io2 + 8 * g for g in range(8)]

    @plsc.parallel_loop(0, D_MODEL, unroll=8)
    def _(d):
        for g in range(8):
            vec = src_v[d, pl.ds(16 * g, 16)]
            plsc.store_scatter(dst_v, [row_g[g], iop64 + d], vec)


@functools.partial(
    pl.kernel,
    mesh=_mesh,
    out_type=jax.ShapeDtypeStruct((VOCAB // 2, DP), jnp.float32),
    scratch_types=[
        pltpu.VMEM((D_MODEL, CH_T), jnp.float32),
        pltpu.VMEM((CH_T // 2, DP), jnp.float32),
    ],
    compiler_params=pltpu.CompilerParams(
        use_tc_tiling_on_sc=True, needs_layout_passes=False
    ),
)
def _table_transpose(tab_t_hbm, rem_hbm, out_hbm, buf_in, buf_out):
    wid = lax.axis_index("s") * NUM_CORES + lax.axis_index("c")

    def body(k, carry):
        c = wid + k * NUM_WORKERS

        @pl.when(c < NFULL)
        def _():
            v0 = pl.multiple_of(c * CH_T, CH_T)
            pltpu.sync_copy(tab_t_hbm.at[:, pl.ds(v0, CH_T)], buf_in)
            _transpose_block(buf_in, buf_out)
            r0 = pl.multiple_of(c * (CH_T // 2), CH_T // 2)
            pltpu.sync_copy(buf_out, out_hbm.at[pl.ds(r0, CH_T // 2)])

        @pl.when(c == NFULL)
        def _():
            pltpu.sync_copy(rem_hbm, buf_out.at[pl.ds(0, REM // 2)])
            pltpu.sync_copy(
                buf_out.at[pl.ds(0, REM // 2)],
                out_hbm.at[pl.ds(NFULL * (CH_T // 2), REM // 2)],
            )

        return carry

    lax.fori_loop(0, K_T, body, 0)


COLS = BATCH // NUM_WORKERS
CHUNK = 128
N_SUB = COLS // CHUNK


@functools.partial(
    pl.kernel,
    mesh=_mesh,
    out_type=jax.ShapeDtypeStruct((SEQ_LEN, BATCH, DP), jnp.float32),
    scratch_types=[
        pltpu.VMEM((COLS,), jnp.int32),
        pltpu.VMEM((COLS, D_MODEL), jnp.float32),
        pltpu.SemaphoreType.DMA,
    ],
    compiler_params=pltpu.CompilerParams(use_tc_tiling_on_sc=False),
)
def _embed_gather(idx_hbm, table_hbm, out_hbm, idx_v, rows_v, sem):
    wid = lax.axis_index("s") * NUM_CORES + lax.axis_index("c")
    col0 = wid * COLS

    def body(s, carry):
        pltpu.sync_copy(idx_hbm.at[pl.ds(s * BATCH + col0, COLS)], idx_v)
        copies = [
            pltpu.async_copy(
                table_hbm.at[idx_v.at[pl.ds(k * CHUNK, CHUNK)]],
                rows_v.at[pl.ds(k * CHUNK, CHUNK)],
                sem,
            )
            for k in range(N_SUB)
        ]
        for c in copies:
            c.wait()
        pltpu.sync_copy(
            rows_v, out_hbm.at[s, pl.ds(col0, COLS), pl.ds(0, D_MODEL)]
        )
        return carry

    lax.fori_loop(0, SEQ_LEN, body, 0)


def kernel(input, table):
    rem_p = table[NFULL * CH_T :, :].reshape(REM // 2, DP)
    tbl_packed = _table_transpose(table.T, rem_p)
    out128 = _embed_gather(input.reshape(TOTAL), tbl_packed.reshape(VOCAB, D_MODEL))
    return out128[:, :, :D_MODEL]


# speedup vs baseline: 2.7155x; 1.7503x over previous
import functools

import jax
import jax.numpy as jnp
from jax import lax
from jax.experimental import pallas as pl
from jax.experimental.pallas import tpu as pltpu
from jax.experimental.pallas import tpu_sc as plsc

SEQ_LEN = 50
BATCH = 16384
D_MODEL = 64
DP = 2 * D_MODEL
VOCAB = 1000000
TOTAL = SEQ_LEN * BATCH

NUM_CORES = 2
NUM_SUBCORES = 16
NUM_WORKERS = NUM_CORES * NUM_SUBCORES

_mesh = plsc.VectorSubcoreMesh(core_axis_name="c", subcore_axis_name="s")

CH_T = 384
NFULL = VOCAB // CH_T
REM = VOCAB - NFULL * CH_T
K_T = NFULL // NUM_WORKERS + 1


def _transpose_block(src_v, dst_v):
    iota = lax.iota(jnp.int32, 16)
    w_j = [(iota + j) & 15 for j in range(16)]
    w2_j = [w >> 1 for w in w_j]
    c_j = [(w_j[j] & 1) * D_MODEL + iota for j in range(16)]
    d_idx = [iota + d0 for d0 in range(0, D_MODEL, 16)]

    @plsc.parallel_loop(0, CH_T // 16, unroll=2)
    def _(vb):
        v0 = vb * 16
        r0 = vb * 8
        for j in range(16):
            v_idx = v0 + w_j[j]
            row_j = r0 + w2_j[j]
            for g in range(D_MODEL // 16):
                vec = plsc.load_gather(src_v, [d_idx[g], v_idx])
                plsc.store_scatter(dst_v, [row_j, c_j[j] + 16 * g], vec)


@functools.partial(
    pl.kernel,
    mesh=_mesh,
    out_type=jax.ShapeDtypeStruct((VOCAB // 2, DP), jnp.float32),
    scratch_types=[
        pltpu.VMEM((D_MODEL, CH_T), jnp.float32),
        pltpu.VMEM((D_MODEL, CH_T), jnp.float32),
        pltpu.VMEM((CH_T // 2, DP), jnp.float32),
        pltpu.VMEM((CH_T // 2, DP), jnp.float32),
        pltpu.SemaphoreType.DMA,
        pltpu.SemaphoreType.DMA,
        pltpu.SemaphoreType.DMA,
        pltpu.SemaphoreType.DMA,
    ],
    compiler_params=pltpu.CompilerParams(
        use_tc_tiling_on_sc=True, needs_layout_passes=False
    ),
)
def _table_transpose(
    tab_t_hbm, rem_hbm, out_hbm, in0, in1, out0, out1, sr0, sr1, sw0, sw1
):
    wid = lax.axis_index("s") * NUM_CORES + lax.axis_index("c")
    bufs_in = (in0, in1)
    bufs_out = (out0, out1)
    sems_r = (sr0, sr1)
    sems_w = (sw0, sw1)

    def read(c, b):
        v0 = pl.multiple_of(c * CH_T, CH_T)
        return pltpu.make_async_copy(
            tab_t_hbm.at[:, pl.ds(v0, CH_T)], bufs_in[b], sems_r[b]
        )

    def write(c, b):
        r0 = pl.multiple_of(c * (CH_T // 2), CH_T // 2)
        return pltpu.make_async_copy(
            bufs_out[b], out_hbm.at[pl.ds(r0, CH_T // 2)], sems_w[b]
        )

    @pl.when(wid < NFULL)
    def _():
        read(wid, 0).start()

    def body(k2, carry):
        for b in range(2):
            k = 2 * k2 + b
            c = wid + k * NUM_WORKERS

            @pl.when(c < NFULL)
            def _():
                read(c, b).wait()
                c_next = c + NUM_WORKERS

                @pl.when(c_next < NFULL)
                def _():
                    read(c_next, 1 - b).start()

                @pl.when(k2 >= 1)
                def _():
                    write(c, b).wait()

                _transpose_block(bufs_in[b], bufs_out[b])
                write(c, b).start()

        return carry

    lax.fori_loop(0, K_T // 2, body, 0)
    write(0, 0).wait()
    write(0, 1).wait()

    @pl.when(wid == NFULL % NUM_WORKERS)
    def _():
        pltpu.sync_copy(rem_hbm, out0.at[pl.ds(0, REM // 2)])
        pltpu.sync_copy(
            out0.at[pl.ds(0, REM // 2)],
            out_hbm.at[pl.ds(NFULL * (CH_T // 2), REM // 2)],
        )


COLS = BATCH // NUM_WORKERS
CHUNK = 128
N_SUB = COLS // CHUNK


@functools.partial(
    pl.kernel,
    mesh=_mesh,
    out_type=jax.ShapeDtypeStruct((SEQ_LEN, BATCH, DP), jnp.float32),
    scratch_types=[
        pltpu.VMEM((COLS,), jnp.int32),
        pltpu.VMEM((COLS, D_MODEL), jnp.float32),
        pltpu.SemaphoreType.DMA,
    ],
    compiler_params=pltpu.CompilerParams(use_tc_tiling_on_sc=False),
)
def _embed_gather(idx_hbm, table_hbm, out_hbm, idx_v, rows_v, sem):
    wid = lax.axis_index("s") * NUM_CORES + lax.axis_index("c")
    col0 = wid * COLS

    def body(s, carry):
        pltpu.sync_copy(idx_hbm.at[pl.ds(s * BATCH + col0, COLS)], idx_v)
        copies = [
            pltpu.async_copy(
                table_hbm.at[idx_v.at[pl.ds(k * CHUNK, CHUNK)]],
                rows_v.at[pl.ds(k * CHUNK, CHUNK)],
                sem,
            )
            for k in range(N_SUB)
        ]
        for c in copies:
            c.wait()
        pltpu.sync_copy(
            rows_v, out_hbm.at[s, pl.ds(col0, COLS), pl.ds(0, D_MODEL)]
        )
        return carry

    lax.fori_loop(0, SEQ_LEN, body, 0)


def kernel(input, table):
    rem_p = table[NFULL * CH_T :, :].reshape(REM // 2, DP)
    tbl_packed = _table_transpose(table.T, rem_p)
    out128 = _embed_gather(input.reshape(TOTAL), tbl_packed.reshape(VOCAB, D_MODEL))
    return out128[:, :, :D_MODEL]


# speedup vs baseline: 4.3608x; 1.6059x over previous
import functools

import jax
import jax.numpy as jnp
from jax import lax
from jax.experimental import pallas as pl
from jax.experimental.pallas import tpu as pltpu
from jax.experimental.pallas import tpu_sc as plsc

SEQ_LEN = 50
BATCH = 16384
D_MODEL = 64
DP = 2 * D_MODEL
VOCAB = 1000000
TOTAL = SEQ_LEN * BATCH

NUM_CORES = 2
NUM_SUBCORES = 16
NUM_WORKERS = NUM_CORES * NUM_SUBCORES

_mesh = plsc.VectorSubcoreMesh(core_axis_name="c", subcore_axis_name="s")

CH_T = 384
NFULL = VOCAB // CH_T
REM = VOCAB - NFULL * CH_T
K_T = NFULL // NUM_WORKERS + 1


def _transpose_block(src_v, dst_v):
    iota = lax.iota(jnp.int32, 16)
    w_j = [(iota + j) & 15 for j in range(16)]
    w2_j = [w >> 1 for w in w_j]
    c_j = [(w_j[j] & 1) * D_MODEL + iota for j in range(16)]
    d_idx = [iota + d0 for d0 in range(0, D_MODEL, 16)]

    @plsc.parallel_loop(0, CH_T // 16, unroll=4)
    def _(vb):
        v0 = vb * 16
        r0 = vb * 8
        for j in range(16):
            v_idx = v0 + w_j[j]
            row_j = r0 + w2_j[j]
            for g in range(D_MODEL // 16):
                vec = plsc.load_gather(src_v, [d_idx[g], v_idx])
                plsc.store_scatter(dst_v, [row_j, c_j[j] + 16 * g], vec)


@functools.partial(
    pl.kernel,
    mesh=_mesh,
    out_type=jax.ShapeDtypeStruct((VOCAB // 2, DP), jnp.float32),
    scratch_types=[
        pltpu.VMEM((D_MODEL, CH_T), jnp.float32),
        pltpu.VMEM((D_MODEL, CH_T), jnp.float32),
        pltpu.VMEM((CH_T // 2, DP), jnp.float32),
        pltpu.VMEM((CH_T // 2, DP), jnp.float32),
        pltpu.SemaphoreType.DMA,
        pltpu.SemaphoreType.DMA,
        pltpu.SemaphoreType.DMA,
        pltpu.SemaphoreType.DMA,
    ],
    compiler_params=pltpu.CompilerParams(
        use_tc_tiling_on_sc=True, needs_layout_passes=False
    ),
)
def _table_transpose(
    tab_t_hbm, rem_hbm, out_hbm, in0, in1, out0, out1, sr0, sr1, sw0, sw1
):
    wid = lax.axis_index("s") * NUM_CORES + lax.axis_index("c")
    bufs_in = (in0, in1)
    bufs_out = (out0, out1)
    sems_r = (sr0, sr1)
    sems_w = (sw0, sw1)

    def read(c, b):
        v0 = pl.multiple_of(c * CH_T, CH_T)
        return pltpu.make_async_copy(
            tab_t_hbm.at[:, pl.ds(v0, CH_T)], bufs_in[b], sems_r[b]
        )

    def write(c, b):
        r0 = pl.multiple_of(c * (CH_T // 2), CH_T // 2)
        return pltpu.make_async_copy(
            bufs_out[b], out_hbm.at[pl.ds(r0, CH_T // 2)], sems_w[b]
        )

    @pl.when(wid < NFULL)
    def _():
        read(wid, 0).start()

    def body(k2, carry):
        for b in range(2):
            k = 2 * k2 + b
            c = wid + k * NUM_WORKERS

            @pl.when(c < NFULL)
            def _():
                read(c, b).wait()
                c_next = c + NUM_WORKERS

                @pl.when(c_next < NFULL)
                def _():
                    read(c_next, 1 - b).start()

                @pl.when(k2 >= 1)
                def _():
                    write(c, b).wait()

                _transpose_block(bufs_in[b], bufs_out[b])
                write(c, b).start()

        return carry

    lax.fori_loop(0, K_T // 2, body, 0)
    write(0, 0).wait()
    write(0, 1).wait()

    @pl.when(wid == NFULL % NUM_WORKERS)
    def _():
        pltpu.sync_copy(rem_hbm, out0.at[pl.ds(0, REM // 2)])
        pltpu.sync_copy(
            out0.at[pl.ds(0, REM // 2)],
            out_hbm.at[pl.ds(NFULL * (CH_T // 2), REM // 2)],
        )


COLS = BATCH // NUM_WORKERS
CHUNK = 128
N_SUB = COLS // CHUNK


@functools.partial(
    pl.kernel,
    mesh=_mesh,
    out_type=jax.ShapeDtypeStruct((SEQ_LEN, BATCH, DP), jnp.float32),
    scratch_types=[
        pltpu.VMEM((COLS,), jnp.int32),
        pltpu.VMEM((COLS, D_MODEL), jnp.float32),
        pltpu.SemaphoreType.DMA,
    ],
    compiler_params=pltpu.CompilerParams(use_tc_tiling_on_sc=False),
)
def _embed_gather(idx_hbm, table_hbm, out_hbm, idx_v, rows_v, sem):
    wid = lax.axis_index("s") * NUM_CORES + lax.axis_index("c")
    col0 = wid * COLS

    def body(s, carry):
        pltpu.sync_copy(idx_hbm.at[pl.ds(s * BATCH + col0, COLS)], idx_v)
        copies = [
            pltpu.async_copy(
                table_hbm.at[idx_v.at[pl.ds(k * CHUNK, CHUNK)]],
                rows_v.at[pl.ds(k * CHUNK, CHUNK)],
                sem,
            )
            for k in range(N_SUB)
        ]
        for c in copies:
            c.wait()
        pltpu.sync_copy(
            rows_v, out_hbm.at[s, pl.ds(col0, COLS), pl.ds(0, D_MODEL)]
        )
        return carry

    lax.fori_loop(0, SEQ_LEN, body, 0)


def kernel(input, table):
    rem_p = table[NFULL * CH_T :, :].reshape(REM // 2, DP)
    tbl_packed = _table_transpose(table.T, rem_p)
    out128 = _embed_gather(input.reshape(TOTAL), tbl_packed.reshape(VOCAB, D_MODEL))
    return out128[:, :, :D_MODEL]


# speedup vs baseline: 4.4109x; 1.0115x over previous
import functools

import jax
import jax.numpy as jnp
from jax import lax
from jax.experimental import pallas as pl
from jax.experimental.pallas import tpu as pltpu
from jax.experimental.pallas import tpu_sc as plsc

SEQ_LEN = 50
BATCH = 16384
D_MODEL = 64
DP = 2 * D_MODEL
VOCAB = 1000000
TOTAL = SEQ_LEN * BATCH

NUM_CORES = 2
NUM_SUBCORES = 16
NUM_WORKERS = NUM_CORES * NUM_SUBCORES

_mesh = plsc.VectorSubcoreMesh(core_axis_name="c", subcore_axis_name="s")

CH_T = 384
NFULL = VOCAB // CH_T
REM = VOCAB - NFULL * CH_T
K_T = NFULL // NUM_WORKERS + 1


def _transpose_block(src_v, dst_v):
    iota = lax.iota(jnp.int32, 16)
    w_j = [(iota + j) & 15 for j in range(16)]
    w2_j = [w >> 1 for w in w_j]
    c_j = [(w_j[j] & 1) * D_MODEL + iota for j in range(16)]
    d_idx = [iota + d0 for d0 in range(0, D_MODEL, 16)]

    @plsc.parallel_loop(0, CH_T // 16, unroll=4)
    def _(vb):
        v0 = vb * 16
        r0 = vb * 8
        for j in range(16):
            v_idx = v0 + w_j[j]
            row_j = r0 + w2_j[j]
            for g in range(D_MODEL // 16):
                vec = plsc.load_gather(src_v, [d_idx[g], v_idx])
                plsc.store_scatter(dst_v, [row_j, c_j[j] + 16 * g], vec)


@functools.partial(
    pl.kernel,
    mesh=_mesh,
    out_type=jax.ShapeDtypeStruct((VOCAB // 2, DP), jnp.float32),
    scratch_types=[
        pltpu.VMEM((D_MODEL, CH_T), jnp.float32),
        pltpu.VMEM((D_MODEL, CH_T), jnp.float32),
        pltpu.VMEM((CH_T // 2, DP), jnp.float32),
        pltpu.VMEM((CH_T // 2, DP), jnp.float32),
        pltpu.SemaphoreType.DMA,
        pltpu.SemaphoreType.DMA,
        pltpu.SemaphoreType.DMA,
        pltpu.SemaphoreType.DMA,
    ],
    compiler_params=pltpu.CompilerParams(
        use_tc_tiling_on_sc=True, needs_layout_passes=False
    ),
)
def _table_transpose(
    tab_t_hbm, rem_hbm, out_hbm, in0, in1, out0, out1, sr0, sr1, sw0, sw1
):
    wid = lax.axis_index("s") * NUM_CORES + lax.axis_index("c")
    bufs_in = (in0, in1)
    bufs_out = (out0, out1)
    sems_r = (sr0, sr1)
    sems_w = (sw0, sw1)

    def read(c, b):
        v0 = pl.multiple_of(c * CH_T, CH_T)
        return pltpu.make_async_copy(
            tab_t_hbm.at[:, pl.ds(v0, CH_T)], bufs_in[b], sems_r[b]
        )

    def write(c, b):
        r0 = pl.multiple_of(c * (CH_T // 2), CH_T // 2)
        return pltpu.make_async_copy(
            bufs_out[b], out_hbm.at[pl.ds(r0, CH_T // 2)], sems_w[b]
        )

    @pl.when(wid < NFULL)
    def _():
        read(wid, 0).start()

    def body(k2, carry):
        for b in range(2):
            k = 2 * k2 + b
            c = wid + k * NUM_WORKERS

            @pl.when(c < NFULL)
            def _():
                read(c, b).wait()
                c_next = c + NUM_WORKERS

                @pl.when(c_next < NFULL)
                def _():
                    read(c_next, 1 - b).start()

                @pl.when(k2 >= 1)
                def _():
                    write(c, b).wait()

                _transpose_block(bufs_in[b], bufs_out[b])
                write(c, b).start()

        return carry

    lax.fori_loop(0, K_T // 2, body, 0)
    write(0, 0).wait()
    write(0, 1).wait()

    @pl.when(wid == NFULL % NUM_WORKERS)
    def _():
        pltpu.sync_copy(rem_hbm, out0.at[pl.ds(0, REM // 2)])
        pltpu.sync_copy(
            out0.at[pl.ds(0, REM // 2)],
            out_hbm.at[pl.ds(NFULL * (CH_T // 2), REM // 2)],
        )


COLS = BATCH // NUM_WORKERS
CHUNK = 128
N_SUB = COLS // CHUNK


NBT = BATCH // CHUNK


def _out_transpose_block(rows_q, tbuf):
    iota = lax.iota(jnp.int32, 16)
    w_j = [(iota + j) & 15 for j in range(16)]

    @plsc.parallel_loop(0, CHUNK // 16, unroll=4)
    def _(bb):
        b0 = bb * 16
        b_idx = b0 + iota
        for j in range(16):
            w = w_j[j]
            for d0 in range(0, D_MODEL, 16):
                f = d0 + w
                vec = plsc.load_gather(rows_q, [b_idx, f])
                plsc.store_scatter(tbuf, [f >> 3, f & 7, b_idx], vec)


@functools.partial(
    pl.kernel,
    mesh=_mesh,
    out_type=jax.ShapeDtypeStruct((SEQ_LEN, 8, NBT, 8, CHUNK), jnp.float32),
    scratch_types=[
        pltpu.VMEM((COLS,), jnp.int32),
        pltpu.VMEM((COLS, D_MODEL), jnp.float32),
        pltpu.VMEM((COLS, D_MODEL), jnp.float32),
        pltpu.VMEM((8, 8, CHUNK), jnp.float32),
        pltpu.SemaphoreType.DMA,
        pltpu.SemaphoreType.DMA,
        pltpu.SemaphoreType.DMA,
    ],
    compiler_params=pltpu.CompilerParams(
        use_tc_tiling_on_sc=False, needs_layout_passes=False
    ),
)
def _embed_gather(
    idx_hbm, table_hbm, out_hbm, idx_v, rows0, rows1, tbuf, sg0, sg1, sw
):
    wid = lax.axis_index("s") * NUM_CORES + lax.axis_index("c")
    col0 = wid * COLS
    bt0 = wid * N_SUB
    rows = (rows0, rows1)
    sems = (sg0, sg1)

    def gather(s, b):
        pltpu.sync_copy(idx_hbm.at[pl.ds(s * BATCH + col0, COLS)], idx_v)
        for k in range(N_SUB):
            pltpu.make_async_copy(
                table_hbm.at[idx_v.at[pl.ds(k * CHUNK, CHUNK)]],
                rows[b].at[pl.ds(k * CHUNK, CHUNK)],
                sems[b],
            ).start()

    def gather_wait(b):
        for k in range(N_SUB):
            pltpu.make_async_copy(
                table_hbm.at[idx_v.at[pl.ds(k * CHUNK, CHUNK)]],
                rows[b].at[pl.ds(k * CHUNK, CHUNK)],
                sems[b],
            ).wait()

    def emit(s, b):
        for q in range(N_SUB):
            _out_transpose_block(rows[b].at[pl.ds(q * CHUNK, CHUNK)], tbuf)
            pltpu.sync_copy(tbuf, out_hbm.at[s, :, bt0 + q])

    gather(0, 0)

    def body(s2, carry):
        for b in range(2):
            s = 2 * s2 + b
            gather_wait(b)

            @pl.when(s + 1 < SEQ_LEN)
            def _():
                gather(s + 1, 1 - b)

            emit(s, b)

        return carry

    lax.fori_loop(0, SEQ_LEN // 2, body, 0)


def kernel(input, table):
    rem_p = table[NFULL * CH_T :, :].reshape(REM // 2, DP)
    tbl_packed = _table_transpose(table.T, rem_p)
    out5 = _embed_gather(input.reshape(TOTAL), tbl_packed.reshape(VOCAB, D_MODEL))
    return out5.transpose(0, 2, 4, 1, 3).reshape(SEQ_LEN, BATCH, D_MODEL)


# speedup vs baseline: 4.7940x; 1.0869x over previous
import functools

import jax
import jax.numpy as jnp
from jax import lax
from jax.experimental import pallas as pl
from jax.experimental.pallas import tpu as pltpu
from jax.experimental.pallas import tpu_sc as plsc

SEQ_LEN = 50
BATCH = 16384
D_MODEL = 64
DP = 2 * D_MODEL
VOCAB = 1000000
TOTAL = SEQ_LEN * BATCH

NUM_CORES = 2
NUM_SUBCORES = 16
NUM_WORKERS = NUM_CORES * NUM_SUBCORES

_mesh = plsc.VectorSubcoreMesh(core_axis_name="c", subcore_axis_name="s")

CH_T = 384
NFULL = VOCAB // CH_T
REM = VOCAB - NFULL * CH_T
K_T = NFULL // NUM_WORKERS + 1


def _transpose_block(src_v, dst_v):
    iota = lax.iota(jnp.int32, 16)
    w_j = [(iota + j) & 15 for j in range(16)]
    w2_j = [w >> 1 for w in w_j]
    c_j = [(w_j[j] & 1) * D_MODEL + iota for j in range(16)]
    d_idx = [iota + d0 for d0 in range(0, D_MODEL, 16)]

    @plsc.parallel_loop(0, CH_T // 16, unroll=4)
    def _(vb):
        v0 = vb * 16
        r0 = vb * 8
        for j in range(16):
            v_idx = v0 + w_j[j]
            row_j = r0 + w2_j[j]
            for g in range(D_MODEL // 16):
                vec = plsc.load_gather(src_v, [d_idx[g], v_idx])
                plsc.store_scatter(dst_v, [row_j, c_j[j] + 16 * g], vec)


@functools.partial(
    pl.kernel,
    mesh=_mesh,
    out_type=jax.ShapeDtypeStruct((VOCAB // 2, DP), jnp.float32),
    scratch_types=[
        pltpu.VMEM((D_MODEL, CH_T), jnp.float32),
        pltpu.VMEM((D_MODEL, CH_T), jnp.float32),
        pltpu.VMEM((CH_T // 2, DP), jnp.float32),
        pltpu.VMEM((CH_T // 2, DP), jnp.float32),
        pltpu.SemaphoreType.DMA,
        pltpu.SemaphoreType.DMA,
        pltpu.SemaphoreType.DMA,
        pltpu.SemaphoreType.DMA,
    ],
    compiler_params=pltpu.CompilerParams(
        use_tc_tiling_on_sc=True, needs_layout_passes=False
    ),
)
def _table_transpose(
    tab_t_hbm, rem_hbm, out_hbm, in0, in1, out0, out1, sr0, sr1, sw0, sw1
):
    wid = lax.axis_index("s") * NUM_CORES + lax.axis_index("c")
    bufs_in = (in0, in1)
    bufs_out = (out0, out1)
    sems_r = (sr0, sr1)
    sems_w = (sw0, sw1)

    def read(c, b):
        v0 = pl.multiple_of(c * CH_T, CH_T)
        return pltpu.make_async_copy(
            tab_t_hbm.at[:, pl.ds(v0, CH_T)], bufs_in[b], sems_r[b]
        )

    def write(c, b):
        r0 = pl.multiple_of(c * (CH_T // 2), CH_T // 2)
        return pltpu.make_async_copy(
            bufs_out[b], out_hbm.at[pl.ds(r0, CH_T // 2)], sems_w[b]
        )

    @pl.when(wid < NFULL)
    def _():
        read(wid, 0).start()

    def body(k2, carry):
        for b in range(2):
            k = 2 * k2 + b
            c = wid + k * NUM_WORKERS

            @pl.when(c < NFULL)
            def _():
                read(c, b).wait()
                c_next = c + NUM_WORKERS

                @pl.when(c_next < NFULL)
                def _():
                    read(c_next, 1 - b).start()

                @pl.when(k2 >= 1)
                def _():
                    write(c, b).wait()

                _transpose_block(bufs_in[b], bufs_out[b])
                write(c, b).start()

        return carry

    lax.fori_loop(0, K_T // 2, body, 0)
    write(0, 0).wait()
    write(0, 1).wait()

    @pl.when(wid == NFULL % NUM_WORKERS)
    def _():
        pltpu.sync_copy(rem_hbm, out0.at[pl.ds(0, REM // 2)])
        pltpu.sync_copy(
            out0.at[pl.ds(0, REM // 2)],
            out_hbm.at[pl.ds(NFULL * (CH_T // 2), REM // 2)],
        )


COLS = BATCH // NUM_WORKERS
CHUNK = 128
N_SUB = COLS // CHUNK


NBT = BATCH // CHUNK


def _out_transpose_block(rows_q, tbuf):
    iota = lax.iota(jnp.int32, 16)
    w_j = [(iota + j) & 15 for j in range(16)]

    @plsc.parallel_loop(0, CHUNK // 16, unroll=4)
    def _(bb):
        b0 = bb * 16
        b_idx = b0 + iota
        for j in range(16):
            w = w_j[j]
            for d0 in range(0, D_MODEL, 16):
                f = d0 + w
                vec = plsc.load_gather(rows_q, [b_idx, f])
                plsc.store_scatter(tbuf, [f >> 3, f & 7, b_idx], vec)


@functools.partial(
    pl.kernel,
    mesh=_mesh,
    out_type=jax.ShapeDtypeStruct((SEQ_LEN, 8, NBT, 8, CHUNK), jnp.float32),
    scratch_types=[
        pltpu.VMEM((COLS,), jnp.int32),
        pltpu.VMEM((COLS, D_MODEL), jnp.float32),
        pltpu.VMEM((COLS, D_MODEL), jnp.float32),
        pltpu.VMEM((8, 8, CHUNK), jnp.float32),
        pltpu.VMEM((8, 8, CHUNK), jnp.float32),
        pltpu.SemaphoreType.DMA,
        pltpu.SemaphoreType.DMA,
        pltpu.SemaphoreType.DMA,
        pltpu.SemaphoreType.DMA,
    ],
    compiler_params=pltpu.CompilerParams(
        use_tc_tiling_on_sc=False, needs_layout_passes=False
    ),
)
def _embed_gather(
    idx_hbm, table_hbm, out_hbm, idx_v, rows0, rows1, tb0, tb1, sg0, sg1, sw0, sw1
):
    wid = lax.axis_index("s") * NUM_CORES + lax.axis_index("c")
    col0 = wid * COLS
    bt0 = wid * N_SUB
    rows = (rows0, rows1)
    sems = (sg0, sg1)
    tbufs = (tb0, tb1)
    sws = (sw0, sw1)

    def gather(s, b):
        pltpu.sync_copy(idx_hbm.at[pl.ds(s * BATCH + col0, COLS)], idx_v)
        for k in range(N_SUB):
            pltpu.make_async_copy(
                table_hbm.at[idx_v.at[pl.ds(k * CHUNK, CHUNK)]],
                rows[b].at[pl.ds(k * CHUNK, CHUNK)],
                sems[b],
            ).start()

    def gather_wait(b):
        for k in range(N_SUB):
            pltpu.make_async_copy(
                table_hbm.at[idx_v.at[pl.ds(k * CHUNK, CHUNK)]],
                rows[b].at[pl.ds(k * CHUNK, CHUNK)],
                sems[b],
            ).wait()

    def wcopy(s, q, tb):
        return pltpu.make_async_copy(
            tbufs[tb], out_hbm.at[s, :, bt0 + q], sws[tb]
        )

    def emit(s, b, first):
        for q in range(N_SUB):
            tb = q % 2

            @pl.when(jnp.logical_or(jnp.logical_not(first), q >= 2))
            def _():
                wcopy(s, q, tb).wait()

            _out_transpose_block(rows[b].at[pl.ds(q * CHUNK, CHUNK)], tbufs[tb])
            wcopy(s, q, tb).start()

    gather(0, 0)

    def body(s2, carry):
        for b in range(2):
            s = 2 * s2 + b
            gather_wait(b)

            @pl.when(s + 1 < SEQ_LEN)
            def _():
                gather(s + 1, 1 - b)

            emit(s, b, jnp.logical_and(s2 == 0, b == 0))

        return carry

    lax.fori_loop(0, SEQ_LEN // 2, body, 0)
    wcopy(0, 0, 0).wait()
    wcopy(0, 1, 1).wait()


def kernel(input, table):
    rem_p = table[NFULL * CH_T :, :].reshape(REM // 2, DP)
    tbl_packed = _table_transpose(table.T, rem_p)
    out5 = _embed_gather(input.reshape(TOTAL), tbl_packed.reshape(VOCAB, D_MODEL))
    return out5.transpose(0, 2, 4, 1, 3).reshape(SEQ_LEN, BATCH, D_MODEL)
